# Initial kernel scaffold; baseline (speedup 1.0000x reference)
import jax
import jax.numpy as jnp
from jax.experimental import pallas as pl


def kernel(x, edge_index, edge_attr, batch_index, W_rel, b_rel, W_root, gn_weight, gn_bias, gn_mean_scale):
    raise NotImplementedError("write your pallas kernel here")


# speedup vs baseline: 4.6158x; 4.6158x over previous
import functools

import jax
import jax.numpy as jnp
from jax import lax
from jax.experimental import pallas as pl
from jax.experimental.pallas import tpu as pltpu
import jax.experimental.pallas.tpu_sc as plsc

N = 10000
E = 320000
D = 128
G = 32
EPS = 1e-5

NC = 2
NS = 16
NW = NC * NS
EP = E // NW
K = 80
NCHUNK = EP // K
STRIPE = 624
TAIL = N - NS * STRIPE
ZB = 16

_mesh = plsc.VectorSubcoreMesh(
    core_axis_name="c", subcore_axis_name="s", num_cores=NC, num_subcores=NS)


@functools.partial(
    pl.kernel,
    out_type=jax.ShapeDtypeStruct((NC, N, D), jnp.float32),
    mesh=_mesh,
    scratch_types=[
        pltpu.VMEM((K,), jnp.int32),
        pltpu.VMEM((K,), jnp.int32),
        pltpu.VMEM((K,), jnp.float32),
        pltpu.VMEM((K, D), jnp.float32),
        pltpu.VMEM((ZB, D), jnp.float32),
        pltpu.VMEM_SHARED((N, D), jnp.float32),
        pltpu.SemaphoreType.DMA,
    ],
)
def _sc_aggregate(x_hbm, src_hbm, dst_hbm, attr_hbm, out_hbm,
                  src_v, dst_v, attr_v, rows_v, zero_v, acc_sh, sem):
    c = lax.axis_index("c")
    s = lax.axis_index("s")
    w = s * NC + c
    base = w * EP

    zvec = jnp.zeros((16,), jnp.float32)

    def _zero_row(r, _):
        for j in range(D // 16):
            zero_v[r, pl.ds(j * 16, 16)] = zvec
        return _

    lax.fori_loop(0, ZB, _zero_row, 0)

    def _zero_acc(i, _):
        pltpu.sync_copy(zero_v, acc_sh.at[pl.ds(s * STRIPE + i * ZB, ZB)])
        return _

    lax.fori_loop(0, STRIPE // ZB, _zero_acc, 0)

    @pl.when(s == 0)
    def _zero_tail():
        pltpu.sync_copy(zero_v, acc_sh.at[pl.ds(NS * STRIPE, TAIL)])

    plsc.subcore_barrier()

    def _chunk(i, _):
        off = base + i * K
        pltpu.sync_copy(src_hbm.at[pl.ds(off, K)], src_v)
        pltpu.sync_copy(dst_hbm.at[pl.ds(off, K)], dst_v)
        pltpu.sync_copy(attr_hbm.at[pl.ds(off, K)], attr_v)
        pltpu.async_copy(x_hbm.at[src_v], rows_v, sem).wait()

        def _scale(g, _2):
            av = attr_v[pl.ds(g * 16, 16)]
            for j in range(16):
                a = av[j]
                e = g * 16 + j
                for q in range(D // 16):
                    rows_v[e, pl.ds(q * 16, 16)] = (
                        rows_v[e, pl.ds(q * 16, 16)] * a)
            return _2

        lax.fori_loop(0, K // 16, _scale, 0)
        pltpu.sync_copy(rows_v, acc_sh.at[dst_v], add=True)
        return _

    lax.fori_loop(0, NCHUNK, _chunk, 0)
    plsc.subcore_barrier()

    pltpu.sync_copy(acc_sh.at[pl.ds(s * STRIPE, STRIPE)],
                    out_hbm.at[c, pl.ds(s * STRIPE, STRIPE)])

    @pl.when(s == 0)
    def _copy_tail():
        pltpu.sync_copy(acc_sh.at[pl.ds(NS * STRIPE, TAIL)],
                        out_hbm.at[c, pl.ds(NS * STRIPE, TAIL)])


def _tc_body(x_ref, p_ref, bi_col_ref, bi_row_ref, wrel_t_ref, brel_ref,
             wroot_t_ref, gnw_ref, gnb_ref, gnms_ref, out_ref):
    x = x_ref[...]
    agg = p_ref[0] + p_ref[1]
    h = (jnp.dot(agg, wrel_t_ref[...], preferred_element_type=jnp.float32)
         + brel_ref[...]
         + jnp.dot(x, wroot_t_ref[...], preferred_element_type=jnp.float32))
    h = jnp.maximum(h, 0.0)

    bi_col = bi_col_ref[...]
    bi_row = bi_row_ref[...]
    mt = (lax.broadcasted_iota(jnp.int32, (G, N), 0) == bi_row)
    mt = mt.astype(jnp.float32)
    m = (lax.broadcasted_iota(jnp.int32, (N, G), 1) == bi_col)
    m = m.astype(jnp.float32)

    cnt = jnp.maximum(jnp.sum(mt, axis=1, keepdims=True), 1.0)
    mean = jnp.dot(mt, h, preferred_element_type=jnp.float32) / cnt
    ms = mean * gnms_ref[...]
    out = h - jnp.dot(m, ms, preferred_element_type=jnp.float32)
    var = jnp.dot(mt, out * out, preferred_element_type=jnp.float32) / cnt
    rstd = 1.0 / jnp.sqrt(var + EPS)
    out = out * jnp.dot(m, rstd, preferred_element_type=jnp.float32)
    out_ref[...] = out * gnw_ref[...] + gnb_ref[...]


def kernel(x, edge_index, edge_attr, batch_index, W_rel, b_rel, W_root,
           gn_weight, gn_bias, gn_mean_scale):
    src = edge_index[0]
    dst = edge_index[1]
    partials = _sc_aggregate(x, src, dst, edge_attr)

    bi_col = batch_index.reshape(N, 1)
    bi_row = batch_index.reshape(1, N)
    out = pl.pallas_call(
        _tc_body,
        out_shape=jax.ShapeDtypeStruct((N, D), jnp.float32),
    )(x, partials, bi_col, bi_row, W_rel.T, b_rel.reshape(1, D), W_root.T,
      gn_weight.reshape(1, D), gn_bias.reshape(1, D),
      gn_mean_scale.reshape(1, D))
    return out
